# baseline (speedup 1.0000x reference)
import functools

import jax
import jax.numpy as jnp
from jax import lax
from jax.experimental import pallas as pl
from jax.experimental.pallas import tpu as pltpu
from jax.experimental.pallas import tpu_sc as plsc

B, S, D = 1024, 200, 64
NC, NS = 2, 16
NW = NC * NS
ROWS_PER_W = B // NW
CHUNK = 100
NCHUNK = S // CHUNK
LANES = 16


def _emb_body(x_hbm, tok_hbm, pos_hbm, out_hbm, idx_v, rows_v, pos_v, sem):
    wid = lax.axis_index("s") * NC + lax.axis_index("c")
    base = wid * ROWS_PER_W

    pltpu.sync_copy(pos_hbm, pos_v)

    def row_body(r, carry):
        b = base + r
        pltpu.sync_copy(x_hbm.at[b], idx_v)
        cps = [
            pltpu.async_copy(
                tok_hbm.at[idx_v.at[j]],
                rows_v.at[pl.ds(j * CHUNK, CHUNK)],
                sem,
            )
            for j in range(NCHUNK)
        ]
        for cp in cps:
            cp.wait()

        def add_body(i, c):
            for d in range(D // LANES):
                sl = pl.ds(d * LANES, LANES)
                plsc.addupdate(rows_v.at[i, sl], pos_v[i, sl])
            return c

        lax.fori_loop(0, S, add_body, 0, unroll=2)

        pltpu.sync_copy(rows_v, out_hbm.at[b])
        return carry

    lax.fori_loop(0, ROWS_PER_W, row_body, 0)


@functools.partial(
    pl.kernel,
    out_type=jax.ShapeDtypeStruct((B, S, D), jnp.float32),
    mesh=plsc.VectorSubcoreMesh(core_axis_name="c", subcore_axis_name="s"),
    scratch_types=[
        pltpu.VMEM((NCHUNK, CHUNK), jnp.int32),
        pltpu.VMEM((S, D), jnp.float32),
        pltpu.VMEM((S, D), jnp.float32),
        pltpu.SemaphoreType.DMA,
    ],
    compiler_params=pltpu.CompilerParams(use_tc_tiling_on_sc=False),
)
def _emb(x_hbm, tok_hbm, pos_hbm, out_hbm, idx_v, rows_v, pos_v, sem):
    _emb_body(x_hbm, tok_hbm, pos_hbm, out_hbm, idx_v, rows_v, pos_v, sem)


@jax.jit
def kernel(x, token_table, pos_table):
    x32 = x.astype(jnp.int32).reshape(B, NCHUNK, CHUNK)
    pos = pos_table[:S]
    return _emb(x32, token_table, pos)
